# Initial kernel scaffold; baseline (speedup 1.0000x reference)
import jax
import jax.numpy as jnp
from jax.experimental import pallas as pl


def kernel(x, embed, W, b):
    raise NotImplementedError("write your pallas kernel here")


# speedup vs baseline: 8.4249x; 8.4249x over previous
import functools

import jax
import jax.numpy as jnp
from jax import lax
from jax.experimental import pallas as pl
from jax.experimental.pallas import tpu as pltpu
from jax.experimental.pallas import tpu_sc as plsc

NUM_CORES = 2
NUM_SUBCORES = 16
LANES = 16
NW = NUM_CORES * NUM_SUBCORES

B = 16384
L = 200
DIM = 16

EPW = B // NW
G = 16
GROUPS = EPW // G
ROWS_G = G * L
CHUNK = 128
NCHUNK = ROWS_G // CHUNK

_MAGIC = 12582912.0

_GATHER_DNUMS = lax.GatherDimensionNumbers(
    offset_dims=(), collapsed_slice_dims=(0,), start_index_map=(0,))


def _lane_shuffle(v, idx):
    return lax.gather(v, idx[:, None], _GATHER_DNUMS, slice_sizes=(1,),
                      mode=lax.GatherScatterMode.PROMISE_IN_BOUNDS)


def _sc_body(x_hbm, emb_hbm, wb_hbm, b_hbm, out_hbm,
             idx_v, rows_v, wb_v, b_v, out_v, gsem):
    c = lax.axis_index("c")
    s = lax.axis_index("s")
    wid = c * NUM_SUBCORES + s
    base = wid * EPW

    pltpu.sync_copy(wb_hbm, wb_v)
    pltpu.sync_copy(b_hbm, b_v)
    bvec = b_v[...]
    wvec = wb_v[...]
    lane_iota = lax.iota(jnp.int32, LANES)

    def group_body(g, carry):
        pltpu.sync_copy(x_hbm.at[pl.ds((base + g * G) * L, ROWS_G)], idx_v)
        for k in range(NCHUNK):
            pltpu.async_copy(
                emb_hbm.at[idx_v.at[pl.ds(k * CHUNK, CHUNK)]],
                rows_v.at[pl.ds(k * CHUNK, CHUNK)], gsem)
        for k in range(NCHUNK):
            pltpu.make_async_copy(
                emb_hbm.at[idx_v.at[pl.ds(k * CHUNK, CHUNK)]],
                rows_v.at[pl.ds(k * CHUNK, CHUNK)], gsem).wait()

        res = jnp.zeros((LANES,), jnp.float32)
        for j in range(G):
            rbase = j * L

            def rbody(i, accs):
                a0, a1, a2, a3 = accs
                rb = rbase + i * 8
                a0 = a0 + rows_v[rb + 0]
                a1 = a1 + rows_v[rb + 1]
                a2 = a2 + rows_v[rb + 2]
                a3 = a3 + rows_v[rb + 3]
                a0 = a0 + rows_v[rb + 4]
                a1 = a1 + rows_v[rb + 5]
                a2 = a2 + rows_v[rb + 6]
                a3 = a3 + rows_v[rb + 7]
                return (a0, a1, a2, a3)

            z16 = jnp.zeros((LANES,), jnp.float32)
            a0, a1, a2, a3 = lax.fori_loop(0, L // 8, rbody,
                                           (z16, z16, z16, z16))
            t = ((a0 + a1) + (a2 + a3)) * wvec
            for sh in (8, 4, 2, 1):
                t = t + _lane_shuffle(t, lane_iota ^ sh)
            res = jnp.where(lane_iota == j, t, res)

        z = res * (1.0 / L) + bvec
        p = 1.0 / (1.0 + jnp.exp(-z))
        r = p * 10000.0
        r = (r + _MAGIC) - _MAGIC
        out_v[pl.ds(g * G, G)] = r / 10000.0
        return carry

    lax.fori_loop(0, GROUPS, group_body, 0)
    pltpu.sync_copy(out_v, out_hbm.at[pl.ds(base, EPW)])


@functools.partial(
    pl.kernel,
    out_type=jax.ShapeDtypeStruct((B,), jnp.float32),
    mesh=plsc.VectorSubcoreMesh(core_axis_name="c", subcore_axis_name="s",
                                num_cores=NUM_CORES,
                                num_subcores=NUM_SUBCORES),
    scratch_types=[
        pltpu.VMEM((ROWS_G,), jnp.int32),
        pltpu.VMEM((ROWS_G, LANES), jnp.float32),
        pltpu.VMEM((LANES,), jnp.float32),
        pltpu.VMEM((LANES,), jnp.float32),
        pltpu.VMEM((EPW,), jnp.float32),
        pltpu.SemaphoreType.DMA,
    ],
    compiler_params=pltpu.CompilerParams(use_tc_tiling_on_sc=False),
)
def _sc_kernel(x_hbm, emb_hbm, wb_hbm, b_hbm, out_hbm,
               idx_v, rows_v, wb_v, b_v, out_v, gsem):
    _sc_body(x_hbm, emb_hbm, wb_hbm, b_hbm, out_hbm,
             idx_v, rows_v, wb_v, b_v, out_v, gsem)


@jax.jit
def kernel(x, embed, W, b):
    x_flat = x.reshape(-1).astype(jnp.int32)
    wb = W.reshape(LANES).astype(jnp.float32)
    b16 = jnp.broadcast_to(b, (LANES,)).astype(jnp.float32)
    y = _sc_kernel(x_flat, embed, wb, b16)
    return y.reshape(B, 1)


# speedup vs baseline: 9.7497x; 1.1573x over previous
import functools

import jax
import jax.numpy as jnp
from jax import lax
from jax.experimental import pallas as pl
from jax.experimental.pallas import tpu as pltpu
from jax.experimental.pallas import tpu_sc as plsc

NUM_CORES = 2
NUM_SUBCORES = 16
LANES = 16
NW = NUM_CORES * NUM_SUBCORES

B = 16384
L = 200
DIM = 16

EPW = B // NW
G = 16
GROUPS = EPW // G
ROWS_G = G * L
CHUNK = 128
NCHUNK = ROWS_G // CHUNK

_MAGIC = 12582912.0

_GATHER_DNUMS = lax.GatherDimensionNumbers(
    offset_dims=(), collapsed_slice_dims=(0,), start_index_map=(0,))


def _lane_shuffle(v, idx):
    return lax.gather(v, idx[:, None], _GATHER_DNUMS, slice_sizes=(1,),
                      mode=lax.GatherScatterMode.PROMISE_IN_BOUNDS)


def _sc_body(x_hbm, emb_hbm, wb_hbm, b_hbm, out_hbm,
             idx0, idx1, rows0, rows1, wb_v, b_v, out_v,
             gsem0, gsem1, isem0, isem1):
    c = lax.axis_index("c")
    s = lax.axis_index("s")
    wid = c * NUM_SUBCORES + s
    base = wid * EPW

    pltpu.sync_copy(wb_hbm, wb_v)
    pltpu.sync_copy(b_hbm, b_v)
    bvec = b_v[...]
    wvec = wb_v[...]
    lane_iota = lax.iota(jnp.int32, LANES)

    def idx_start(g, idx_v, isem):
        pltpu.async_copy(x_hbm.at[pl.ds((base + g * G) * L, ROWS_G)],
                         idx_v, isem)

    def idx_wait(idx_v, isem):
        pltpu.make_async_copy(x_hbm.at[pl.ds(0, ROWS_G)], idx_v, isem).wait()

    def fire(idx_v, rows_v, gsem):
        for k in range(NCHUNK):
            pltpu.async_copy(
                emb_hbm.at[idx_v.at[pl.ds(k * CHUNK, CHUNK)]],
                rows_v.at[pl.ds(k * CHUNK, CHUNK)], gsem)

    def drain(idx_v, rows_v, gsem):
        for k in range(NCHUNK):
            pltpu.make_async_copy(
                emb_hbm.at[idx_v.at[pl.ds(k * CHUNK, CHUNK)]],
                rows_v.at[pl.ds(k * CHUNK, CHUNK)], gsem).wait()

    def compute(g, rows_v):
        res = jnp.zeros((LANES,), jnp.float32)
        for j in range(G):
            rbase = j * L

            def rbody(i, accs):
                a0, a1, a2, a3 = accs
                rb = rbase + i * 8
                a0 = a0 + rows_v[rb + 0]
                a1 = a1 + rows_v[rb + 1]
                a2 = a2 + rows_v[rb + 2]
                a3 = a3 + rows_v[rb + 3]
                a0 = a0 + rows_v[rb + 4]
                a1 = a1 + rows_v[rb + 5]
                a2 = a2 + rows_v[rb + 6]
                a3 = a3 + rows_v[rb + 7]
                return (a0, a1, a2, a3)

            z16 = jnp.zeros((LANES,), jnp.float32)
            a0, a1, a2, a3 = lax.fori_loop(0, L // 8, rbody,
                                           (z16, z16, z16, z16))
            t = ((a0 + a1) + (a2 + a3)) * wvec
            for sh in (8, 4, 2, 1):
                t = t + _lane_shuffle(t, lane_iota ^ sh)
            res = jnp.where(lane_iota == j, t, res)

        z = res * (1.0 / L) + bvec
        p = 1.0 / (1.0 + jnp.exp(-z))
        r = p * 10000.0
        r = (r + _MAGIC) - _MAGIC
        out_v[pl.ds(g * G, G)] = r / 10000.0

    pltpu.sync_copy(x_hbm.at[pl.ds(base * L, ROWS_G)], idx0)
    fire(idx0, rows0, gsem0)
    idx_start(1, idx1, isem1)

    def pipe_body(i, carry):
        a = 2 * i
        idx_wait(idx1, isem1)
        fire(idx1, rows1, gsem1)
        drain(idx0, rows0, gsem0)
        idx_start(a + 2, idx0, isem0)
        compute(a, rows0)
        idx_wait(idx0, isem0)
        fire(idx0, rows0, gsem0)
        drain(idx1, rows1, gsem1)
        idx_start(a + 3, idx1, isem1)
        compute(a + 1, rows1)
        return carry

    lax.fori_loop(0, GROUPS // 2 - 1, pipe_body, 0)

    idx_wait(idx1, isem1)
    fire(idx1, rows1, gsem1)
    drain(idx0, rows0, gsem0)
    compute(GROUPS - 2, rows0)
    drain(idx1, rows1, gsem1)
    compute(GROUPS - 1, rows1)

    pltpu.sync_copy(out_v, out_hbm.at[pl.ds(base, EPW)])


@functools.partial(
    pl.kernel,
    out_type=jax.ShapeDtypeStruct((B,), jnp.float32),
    mesh=plsc.VectorSubcoreMesh(core_axis_name="c", subcore_axis_name="s",
                                num_cores=NUM_CORES,
                                num_subcores=NUM_SUBCORES),
    scratch_types=[
        pltpu.VMEM((ROWS_G,), jnp.int32),
        pltpu.VMEM((ROWS_G,), jnp.int32),
        pltpu.VMEM((ROWS_G, LANES), jnp.float32),
        pltpu.VMEM((ROWS_G, LANES), jnp.float32),
        pltpu.VMEM((LANES,), jnp.float32),
        pltpu.VMEM((LANES,), jnp.float32),
        pltpu.VMEM((EPW,), jnp.float32),
        pltpu.SemaphoreType.DMA,
        pltpu.SemaphoreType.DMA,
        pltpu.SemaphoreType.DMA,
        pltpu.SemaphoreType.DMA,
    ],
    compiler_params=pltpu.CompilerParams(use_tc_tiling_on_sc=False),
)
def _sc_kernel(x_hbm, emb_hbm, wb_hbm, b_hbm, out_hbm,
               idx0, idx1, rows0, rows1, wb_v, b_v, out_v,
               gsem0, gsem1, isem0, isem1):
    _sc_body(x_hbm, emb_hbm, wb_hbm, b_hbm, out_hbm,
             idx0, idx1, rows0, rows1, wb_v, b_v, out_v,
             gsem0, gsem1, isem0, isem1)


@jax.jit
def kernel(x, embed, W, b):
    x_flat = x.reshape(-1).astype(jnp.int32)
    wb = W.reshape(LANES).astype(jnp.float32)
    b16 = jnp.broadcast_to(b, (LANES,)).astype(jnp.float32)
    y = _sc_kernel(x_flat, embed, wb, b16)
    return y.reshape(B, 1)


# speedup vs baseline: 19.6895x; 2.0195x over previous
import functools

import jax
import jax.numpy as jnp
from jax import lax
from jax.experimental import pallas as pl
from jax.experimental.pallas import tpu as pltpu
from jax.experimental.pallas import tpu_sc as plsc

NUM_CORES = 2
NUM_SUBCORES = 16
LANES = 16
NW = NUM_CORES * NUM_SUBCORES

B = 16384
L = 200
DIM = 16
VOCAB_SIZE = 1000000

EPW = B // NW
G = 16
GROUPS = EPW // G
ROWS_G = G * L
CHUNK = 128
NCHUNK = ROWS_G // CHUNK

EW_BLK = 4096

_MAGIC = 12582912.0


def _ew_body(emb_ref, w_ref, out_ref):
    out_ref[...] = jnp.sum(emb_ref[...] * w_ref[...], axis=0)


_tc_ew = pl.pallas_call(
    _ew_body,
    out_shape=jax.ShapeDtypeStruct((VOCAB_SIZE,), jnp.float32),
    grid=(pl.cdiv(VOCAB_SIZE, EW_BLK),),
    in_specs=[
        pl.BlockSpec((DIM, EW_BLK), lambda i: (0, i)),
        pl.BlockSpec((DIM, 1), lambda i: (0, 0)),
    ],
    out_specs=pl.BlockSpec((EW_BLK,), lambda i: (i,)),
)


def _sc_body(xt_hbm, ew_hbm, b_hbm, out_hbm,
             slab_v, idx0, idx1, val0, val1, b_v, out_v,
             gsem0, gsem1, ssem):
    c = lax.axis_index("c")
    s = lax.axis_index("s")
    wid = c * NUM_SUBCORES + s
    base = wid * EPW

    pltpu.sync_copy(b_hbm, b_v)
    bvec = b_v[...]

    def slab_start(g):
        pltpu.async_copy(xt_hbm.at[:, pl.ds(base + g * G, G)], slab_v, ssem)

    def slab_wait():
        pltpu.make_async_copy(xt_hbm.at[:, pl.ds(0, G)], slab_v, ssem).wait()

    def repack(idx_v):
        def body(r, carry):
            idx_v[pl.ds(r * LANES, LANES)] = slab_v[r]
            return carry
        lax.fori_loop(0, L, body, 0)

    def fire(idx_v, val_v, gsem):
        for k in range(NCHUNK):
            pltpu.async_copy(
                ew_hbm.at[idx_v.at[pl.ds(k * CHUNK, CHUNK)]],
                val_v.at[pl.ds(k * CHUNK, CHUNK)], gsem)

    def drain(idx_v, val_v, gsem):
        for k in range(NCHUNK):
            pltpu.make_async_copy(
                ew_hbm.at[idx_v.at[pl.ds(k * CHUNK, CHUNK)]],
                val_v.at[pl.ds(k * CHUNK, CHUNK)], gsem).wait()

    def compute(g, val_v):
        def body(i, accs):
            a0, a1 = accs
            rb = i * (2 * LANES)
            a0 = a0 + val_v[pl.ds(rb, LANES)]
            a1 = a1 + val_v[pl.ds(rb + LANES, LANES)]
            return (a0, a1)

        z16 = jnp.zeros((LANES,), jnp.float32)
        a0, a1 = lax.fori_loop(0, L // 2, body, (z16, z16))
        z = (a0 + a1) * (1.0 / L) + bvec
        p = 1.0 / (1.0 + jnp.exp(-z))
        r = p * 10000.0
        r = (r + _MAGIC) - _MAGIC
        out_v[pl.ds(g * G, G)] = r / 10000.0

    pltpu.sync_copy(xt_hbm.at[:, pl.ds(base, G)], slab_v)
    repack(idx0)
    fire(idx0, val0, gsem0)
    slab_start(1)

    def pipe_body(i, carry):
        a = 2 * i
        slab_wait()
        repack(idx1)
        fire(idx1, val1, gsem1)
        slab_start(a + 2)
        drain(idx0, val0, gsem0)
        compute(a, val0)
        slab_wait()
        repack(idx0)
        fire(idx0, val0, gsem0)
        slab_start(a + 3)
        drain(idx1, val1, gsem1)
        compute(a + 1, val1)
        return carry

    lax.fori_loop(0, GROUPS // 2 - 1, pipe_body, 0)

    slab_wait()
    repack(idx1)
    fire(idx1, val1, gsem1)
    drain(idx0, val0, gsem0)
    compute(GROUPS - 2, val0)
    drain(idx1, val1, gsem1)
    compute(GROUPS - 1, val1)

    pltpu.sync_copy(out_v, out_hbm.at[pl.ds(base, EPW)])


@functools.partial(
    pl.kernel,
    out_type=jax.ShapeDtypeStruct((B,), jnp.float32),
    mesh=plsc.VectorSubcoreMesh(core_axis_name="c", subcore_axis_name="s",
                                num_cores=NUM_CORES,
                                num_subcores=NUM_SUBCORES),
    scratch_types=[
        pltpu.VMEM((L, G), jnp.int32),
        pltpu.VMEM((ROWS_G,), jnp.int32),
        pltpu.VMEM((ROWS_G,), jnp.int32),
        pltpu.VMEM((ROWS_G,), jnp.float32),
        pltpu.VMEM((ROWS_G,), jnp.float32),
        pltpu.VMEM((LANES,), jnp.float32),
        pltpu.VMEM((EPW,), jnp.float32),
        pltpu.SemaphoreType.DMA,
        pltpu.SemaphoreType.DMA,
        pltpu.SemaphoreType.DMA,
    ],
    compiler_params=pltpu.CompilerParams(use_tc_tiling_on_sc=False),
)
def _sc_kernel(xt_hbm, ew_hbm, b_hbm, out_hbm,
               slab_v, idx0, idx1, val0, val1, b_v, out_v,
               gsem0, gsem1, ssem):
    _sc_body(xt_hbm, ew_hbm, b_hbm, out_hbm,
             slab_v, idx0, idx1, val0, val1, b_v, out_v,
             gsem0, gsem1, ssem)


@jax.jit
def kernel(x, embed, W, b):
    emb_t = embed.T
    ew = _tc_ew(emb_t, W.astype(jnp.float32))
    xt = x.T.astype(jnp.int32)
    b16 = jnp.broadcast_to(b, (LANES,)).astype(jnp.float32)
    y = _sc_kernel(xt, ew, b16)
    return y.reshape(B, 1)


# speedup vs baseline: 19.7256x; 1.0018x over previous
import functools

import jax
import jax.numpy as jnp
from jax import lax
from jax.experimental import pallas as pl
from jax.experimental.pallas import tpu as pltpu
from jax.experimental.pallas import tpu_sc as plsc

NUM_CORES = 2
NUM_SUBCORES = 16
LANES = 16
NW = NUM_CORES * NUM_SUBCORES

B = 16384
L = 200
DIM = 16
VOCAB_SIZE = 1000000

EPW = B // NW
G = 16
GROUPS = EPW // G
ROWS_G = G * L
CHUNK = 3200
NCHUNK = ROWS_G // CHUNK

EW_BLK = 4096

_MAGIC = 12582912.0


def _ew_body(emb_ref, w_ref, out_ref):
    out_ref[...] = jnp.sum(emb_ref[...] * w_ref[...], axis=0)


_tc_ew = pl.pallas_call(
    _ew_body,
    out_shape=jax.ShapeDtypeStruct((VOCAB_SIZE,), jnp.float32),
    grid=(pl.cdiv(VOCAB_SIZE, EW_BLK),),
    in_specs=[
        pl.BlockSpec((DIM, EW_BLK), lambda i: (0, i)),
        pl.BlockSpec((DIM, 1), lambda i: (0, 0)),
    ],
    out_specs=pl.BlockSpec((EW_BLK,), lambda i: (i,)),
)


def _sc_body(xt_hbm, ew_hbm, b_hbm, out_hbm,
             slab_v, idx0, idx1, val0, val1, b_v, out_v,
             gsem0, gsem1, ssem):
    c = lax.axis_index("c")
    s = lax.axis_index("s")
    wid = c * NUM_SUBCORES + s
    base = wid * EPW

    pltpu.sync_copy(b_hbm, b_v)
    bvec = b_v[...]

    def slab_start(g):
        pltpu.async_copy(xt_hbm.at[:, pl.ds(base + g * G, G)], slab_v, ssem)

    def slab_wait():
        pltpu.make_async_copy(xt_hbm.at[:, pl.ds(0, G)], slab_v, ssem).wait()

    def repack(idx_v):
        def body(r, carry):
            idx_v[pl.ds(r * LANES, LANES)] = slab_v[r]
            return carry
        lax.fori_loop(0, L, body, 0)

    def fire(idx_v, val_v, gsem):
        for k in range(NCHUNK):
            pltpu.async_copy(
                ew_hbm.at[idx_v.at[pl.ds(k * CHUNK, CHUNK)]],
                val_v.at[pl.ds(k * CHUNK, CHUNK)], gsem)

    def drain(idx_v, val_v, gsem):
        for k in range(NCHUNK):
            pltpu.make_async_copy(
                ew_hbm.at[idx_v.at[pl.ds(k * CHUNK, CHUNK)]],
                val_v.at[pl.ds(k * CHUNK, CHUNK)], gsem).wait()

    def compute(g, val_v):
        def body(i, accs):
            a0, a1 = accs
            rb = i * (2 * LANES)
            a0 = a0 + val_v[pl.ds(rb, LANES)]
            a1 = a1 + val_v[pl.ds(rb + LANES, LANES)]
            return (a0, a1)

        z16 = jnp.zeros((LANES,), jnp.float32)
        a0, a1 = lax.fori_loop(0, L // 2, body, (z16, z16))
        z = (a0 + a1) * (1.0 / L) + bvec
        p = 1.0 / (1.0 + jnp.exp(-z))
        r = p * 10000.0
        r = (r + _MAGIC) - _MAGIC
        out_v[pl.ds(g * G, G)] = r / 10000.0

    pltpu.sync_copy(xt_hbm.at[:, pl.ds(base, G)], slab_v)
    repack(idx0)
    fire(idx0, val0, gsem0)
    slab_start(1)

    def pipe_body(i, carry):
        a = 2 * i
        slab_wait()
        repack(idx1)
        fire(idx1, val1, gsem1)
        slab_start(a + 2)
        drain(idx0, val0, gsem0)
        compute(a, val0)
        slab_wait()
        repack(idx0)
        fire(idx0, val0, gsem0)
        slab_start(a + 3)
        drain(idx1, val1, gsem1)
        compute(a + 1, val1)
        return carry

    lax.fori_loop(0, GROUPS // 2 - 1, pipe_body, 0)

    slab_wait()
    repack(idx1)
    fire(idx1, val1, gsem1)
    drain(idx0, val0, gsem0)
    compute(GROUPS - 2, val0)
    drain(idx1, val1, gsem1)
    compute(GROUPS - 1, val1)

    pltpu.sync_copy(out_v, out_hbm.at[pl.ds(base, EPW)])


@functools.partial(
    pl.kernel,
    out_type=jax.ShapeDtypeStruct((B,), jnp.float32),
    mesh=plsc.VectorSubcoreMesh(core_axis_name="c", subcore_axis_name="s",
                                num_cores=NUM_CORES,
                                num_subcores=NUM_SUBCORES),
    scratch_types=[
        pltpu.VMEM((L, G), jnp.int32),
        pltpu.VMEM((ROWS_G,), jnp.int32),
        pltpu.VMEM((ROWS_G,), jnp.int32),
        pltpu.VMEM((ROWS_G,), jnp.float32),
        pltpu.VMEM((ROWS_G,), jnp.float32),
        pltpu.VMEM((LANES,), jnp.float32),
        pltpu.VMEM((EPW,), jnp.float32),
        pltpu.SemaphoreType.DMA,
        pltpu.SemaphoreType.DMA,
        pltpu.SemaphoreType.DMA,
    ],
    compiler_params=pltpu.CompilerParams(use_tc_tiling_on_sc=False),
)
def _sc_kernel(xt_hbm, ew_hbm, b_hbm, out_hbm,
               slab_v, idx0, idx1, val0, val1, b_v, out_v,
               gsem0, gsem1, ssem):
    _sc_body(xt_hbm, ew_hbm, b_hbm, out_hbm,
             slab_v, idx0, idx1, val0, val1, b_v, out_v,
             gsem0, gsem1, ssem)


@jax.jit
def kernel(x, embed, W, b):
    emb_t = embed.T
    ew = _tc_ew(emb_t, W.astype(jnp.float32))
    xt = x.T.astype(jnp.int32)
    b16 = jnp.broadcast_to(b, (LANES,)).astype(jnp.float32)
    y = _sc_kernel(xt, ew, b16)
    return y.reshape(B, 1)


# speedup vs baseline: 26.2945x; 1.3330x over previous
import functools

import jax
import jax.numpy as jnp
from jax import lax
from jax.experimental import pallas as pl
from jax.experimental.pallas import tpu as pltpu
from jax.experimental.pallas import tpu_sc as plsc

NUM_CORES = 2
NUM_SUBCORES = 16
LANES = 16
NW = NUM_CORES * NUM_SUBCORES

B = 16384
L = 200
DIM = 16
VOCAB_SIZE = 1000000

EPW = B // NW
G = 16
GROUPS = EPW // G
ROWS_G = G * L
CHUNK = 3200
NCHUNK = ROWS_G // CHUNK

EW_BLK = 4096

_MAGIC = 12582912.0


def _ew_body(emb_ref, w_ref, out_ref):
    out_ref[...] = jnp.sum(emb_ref[...] * w_ref[...], axis=0)


_tc_ew = pl.pallas_call(
    _ew_body,
    out_shape=jax.ShapeDtypeStruct((VOCAB_SIZE,), jnp.float32),
    grid=(pl.cdiv(VOCAB_SIZE, EW_BLK),),
    in_specs=[
        pl.BlockSpec((DIM, EW_BLK), lambda i: (0, i)),
        pl.BlockSpec((DIM, 1), lambda i: (0, 0)),
    ],
    out_specs=pl.BlockSpec((EW_BLK,), lambda i: (i,)),
)


def _sc_body(xt_hbm, ew_hbm, b_hbm, out_hbm,
             slab_v, idx0, idx1, val0, val1, b_v, out_v, ew_sh,
             gsem0, gsem1, ssem):
    c = lax.axis_index("c")
    s = lax.axis_index("s")
    wid = c * NUM_SUBCORES + s
    base = wid * EPW

    @pl.when(s == 0)
    def _():
        pltpu.sync_copy(ew_hbm, ew_sh)
    pltpu.sync_copy(b_hbm, b_v)
    bvec = b_v[...]
    plsc.subcore_barrier()

    def slab_start(g):
        pltpu.async_copy(xt_hbm.at[:, pl.ds(base + g * G, G)], slab_v, ssem)

    def slab_wait():
        pltpu.make_async_copy(xt_hbm.at[:, pl.ds(0, G)], slab_v, ssem).wait()

    def repack(idx_v):
        def body(r, carry):
            idx_v[pl.ds(r * LANES, LANES)] = slab_v[r]
            return carry
        lax.fori_loop(0, L, body, 0)

    def fire(idx_v, val_v, gsem):
        for k in range(NCHUNK):
            pltpu.async_copy(
                ew_sh.at[idx_v.at[pl.ds(k * CHUNK, CHUNK)]],
                val_v.at[pl.ds(k * CHUNK, CHUNK)], gsem)

    def drain(idx_v, val_v, gsem):
        for k in range(NCHUNK):
            pltpu.make_async_copy(
                ew_sh.at[idx_v.at[pl.ds(k * CHUNK, CHUNK)]],
                val_v.at[pl.ds(k * CHUNK, CHUNK)], gsem).wait()

    def compute(g, val_v):
        def body(i, accs):
            a0, a1 = accs
            rb = i * (2 * LANES)
            a0 = a0 + val_v[pl.ds(rb, LANES)]
            a1 = a1 + val_v[pl.ds(rb + LANES, LANES)]
            return (a0, a1)

        z16 = jnp.zeros((LANES,), jnp.float32)
        a0, a1 = lax.fori_loop(0, L // 2, body, (z16, z16))
        z = (a0 + a1) * (1.0 / L) + bvec
        p = 1.0 / (1.0 + jnp.exp(-z))
        r = p * 10000.0
        r = (r + _MAGIC) - _MAGIC
        out_v[pl.ds(g * G, G)] = r / 10000.0

    pltpu.sync_copy(xt_hbm.at[:, pl.ds(base, G)], slab_v)
    repack(idx0)
    fire(idx0, val0, gsem0)
    slab_start(1)

    def pipe_body(i, carry):
        a = 2 * i
        slab_wait()
        repack(idx1)
        fire(idx1, val1, gsem1)
        slab_start(a + 2)
        drain(idx0, val0, gsem0)
        compute(a, val0)
        slab_wait()
        repack(idx0)
        fire(idx0, val0, gsem0)
        slab_start(a + 3)
        drain(idx1, val1, gsem1)
        compute(a + 1, val1)
        return carry

    lax.fori_loop(0, GROUPS // 2 - 1, pipe_body, 0)

    slab_wait()
    repack(idx1)
    fire(idx1, val1, gsem1)
    drain(idx0, val0, gsem0)
    compute(GROUPS - 2, val0)
    drain(idx1, val1, gsem1)
    compute(GROUPS - 1, val1)

    pltpu.sync_copy(out_v, out_hbm.at[pl.ds(base, EPW)])


@functools.partial(
    pl.kernel,
    out_type=jax.ShapeDtypeStruct((B,), jnp.float32),
    mesh=plsc.VectorSubcoreMesh(core_axis_name="c", subcore_axis_name="s",
                                num_cores=NUM_CORES,
                                num_subcores=NUM_SUBCORES),
    scratch_types=[
        pltpu.VMEM((L, G), jnp.int32),
        pltpu.VMEM((ROWS_G,), jnp.int32),
        pltpu.VMEM((ROWS_G,), jnp.int32),
        pltpu.VMEM((ROWS_G,), jnp.float32),
        pltpu.VMEM((ROWS_G,), jnp.float32),
        pltpu.VMEM((LANES,), jnp.float32),
        pltpu.VMEM((EPW,), jnp.float32),
        pltpu.VMEM_SHARED((VOCAB_SIZE,), jnp.float32),
        pltpu.SemaphoreType.DMA,
        pltpu.SemaphoreType.DMA,
        pltpu.SemaphoreType.DMA,
    ],
    compiler_params=pltpu.CompilerParams(use_tc_tiling_on_sc=False),
)
def _sc_kernel(xt_hbm, ew_hbm, b_hbm, out_hbm,
               slab_v, idx0, idx1, val0, val1, b_v, out_v, ew_sh,
               gsem0, gsem1, ssem):
    _sc_body(xt_hbm, ew_hbm, b_hbm, out_hbm,
             slab_v, idx0, idx1, val0, val1, b_v, out_v, ew_sh,
             gsem0, gsem1, ssem)


@jax.jit
def kernel(x, embed, W, b):
    emb_t = embed.T
    ew = _tc_ew(emb_t, W.astype(jnp.float32))
    xt = x.T.astype(jnp.int32)
    b16 = jnp.broadcast_to(b, (LANES,)).astype(jnp.float32)
    y = _sc_kernel(xt, ew, b16)
    return y.reshape(B, 1)


# speedup vs baseline: 29.1151x; 1.1073x over previous
import functools

import jax
import jax.numpy as jnp
from jax import lax
from jax.experimental import pallas as pl
from jax.experimental.pallas import tpu as pltpu
from jax.experimental.pallas import tpu_sc as plsc

NUM_CORES = 2
NUM_SUBCORES = 16
LANES = 16
NW = NUM_CORES * NUM_SUBCORES

B = 16384
L = 200
DIM = 16
VOCAB_SIZE = 1000000

EPW = B // NW
G = 16
GROUPS = EPW // G
ROWS_G = G * L
CHUNK = 3200
NCHUNK = ROWS_G // CHUNK

EW_BLK = 4096

_MAGIC = 12582912.0


def _ew_body(emb_ref, w_ref, out_ref):
    out_ref[...] = jnp.sum(emb_ref[...] * w_ref[...], axis=0)


_tc_ew = pl.pallas_call(
    _ew_body,
    out_shape=jax.ShapeDtypeStruct((VOCAB_SIZE,), jnp.float32),
    grid=(pl.cdiv(VOCAB_SIZE, EW_BLK),),
    in_specs=[
        pl.BlockSpec((DIM, EW_BLK), lambda i: (0, i)),
        pl.BlockSpec((DIM, 1), lambda i: (0, 0)),
    ],
    out_specs=pl.BlockSpec((EW_BLK,), lambda i: (i,)),
)


def _sc_body(xt_hbm, ew_hbm, b_hbm, out_hbm,
             slab_v, idx0, idx1, val0, val1, b_v, out_v, ew_sh,
             gsem0, gsem1, ssem):
    c = lax.axis_index("c")
    s = lax.axis_index("s")
    wid = c * NUM_SUBCORES + s
    base = wid * EPW

    @pl.when(s == 0)
    def _():
        pltpu.sync_copy(ew_hbm, ew_sh)
    pltpu.sync_copy(b_hbm, b_v)
    bvec = b_v[...]
    plsc.subcore_barrier()

    def slab_start(g):
        pltpu.async_copy(xt_hbm.at[:, pl.ds(base + g * G, G)], slab_v, ssem)

    def slab_wait():
        pltpu.make_async_copy(xt_hbm.at[:, pl.ds(0, G)], slab_v, ssem).wait()

    def repack(idx_v):
        def body(r, carry):
            rb = r * 8
            for k in range(8):
                idx_v[pl.ds((rb + k) * LANES, LANES)] = slab_v[rb + k]
            return carry
        lax.fori_loop(0, L // 8, body, 0)

    def fire(idx_v, val_v, gsem):
        for k in range(NCHUNK):
            pltpu.async_copy(
                ew_sh.at[idx_v.at[pl.ds(k * CHUNK, CHUNK)]],
                val_v.at[pl.ds(k * CHUNK, CHUNK)], gsem)

    def drain(idx_v, val_v, gsem):
        for k in range(NCHUNK):
            pltpu.make_async_copy(
                ew_sh.at[idx_v.at[pl.ds(k * CHUNK, CHUNK)]],
                val_v.at[pl.ds(k * CHUNK, CHUNK)], gsem).wait()

    def compute(g, val_v):
        def body(i, accs):
            a0, a1, a2, a3 = accs
            rb = i * (8 * LANES)
            a0 = a0 + val_v[pl.ds(rb, LANES)]
            a1 = a1 + val_v[pl.ds(rb + LANES, LANES)]
            a2 = a2 + val_v[pl.ds(rb + 2 * LANES, LANES)]
            a3 = a3 + val_v[pl.ds(rb + 3 * LANES, LANES)]
            a0 = a0 + val_v[pl.ds(rb + 4 * LANES, LANES)]
            a1 = a1 + val_v[pl.ds(rb + 5 * LANES, LANES)]
            a2 = a2 + val_v[pl.ds(rb + 6 * LANES, LANES)]
            a3 = a3 + val_v[pl.ds(rb + 7 * LANES, LANES)]
            return (a0, a1, a2, a3)

        z16 = jnp.zeros((LANES,), jnp.float32)
        a0, a1, a2, a3 = lax.fori_loop(0, L // 8, body,
                                       (z16, z16, z16, z16))
        z = ((a0 + a1) + (a2 + a3)) * (1.0 / L) + bvec
        p = 1.0 / (1.0 + jnp.exp(-z))
        r = p * 10000.0
        r = (r + _MAGIC) - _MAGIC
        out_v[pl.ds(g * G, G)] = r / 10000.0

    pltpu.sync_copy(xt_hbm.at[:, pl.ds(base, G)], slab_v)
    repack(idx0)
    fire(idx0, val0, gsem0)
    slab_start(1)

    def pipe_body(i, carry):
        a = 2 * i
        slab_wait()
        repack(idx1)
        fire(idx1, val1, gsem1)
        slab_start(a + 2)
        drain(idx0, val0, gsem0)
        compute(a, val0)
        slab_wait()
        repack(idx0)
        fire(idx0, val0, gsem0)
        slab_start(a + 3)
        drain(idx1, val1, gsem1)
        compute(a + 1, val1)
        return carry

    lax.fori_loop(0, GROUPS // 2 - 1, pipe_body, 0)

    slab_wait()
    repack(idx1)
    fire(idx1, val1, gsem1)
    drain(idx0, val0, gsem0)
    compute(GROUPS - 2, val0)
    drain(idx1, val1, gsem1)
    compute(GROUPS - 1, val1)

    pltpu.sync_copy(out_v, out_hbm.at[pl.ds(base, EPW)])


@functools.partial(
    pl.kernel,
    out_type=jax.ShapeDtypeStruct((B,), jnp.float32),
    mesh=plsc.VectorSubcoreMesh(core_axis_name="c", subcore_axis_name="s",
                                num_cores=NUM_CORES,
                                num_subcores=NUM_SUBCORES),
    scratch_types=[
        pltpu.VMEM((L, G), jnp.int32),
        pltpu.VMEM((ROWS_G,), jnp.int32),
        pltpu.VMEM((ROWS_G,), jnp.int32),
        pltpu.VMEM((ROWS_G,), jnp.float32),
        pltpu.VMEM((ROWS_G,), jnp.float32),
        pltpu.VMEM((LANES,), jnp.float32),
        pltpu.VMEM((EPW,), jnp.float32),
        pltpu.VMEM_SHARED((VOCAB_SIZE,), jnp.float32),
        pltpu.SemaphoreType.DMA,
        pltpu.SemaphoreType.DMA,
        pltpu.SemaphoreType.DMA,
    ],
    compiler_params=pltpu.CompilerParams(use_tc_tiling_on_sc=False),
)
def _sc_kernel(xt_hbm, ew_hbm, b_hbm, out_hbm,
               slab_v, idx0, idx1, val0, val1, b_v, out_v, ew_sh,
               gsem0, gsem1, ssem):
    _sc_body(xt_hbm, ew_hbm, b_hbm, out_hbm,
             slab_v, idx0, idx1, val0, val1, b_v, out_v, ew_sh,
             gsem0, gsem1, ssem)


@jax.jit
def kernel(x, embed, W, b):
    emb_t = embed.T
    ew = _tc_ew(emb_t, W.astype(jnp.float32))
    xt = x.T.astype(jnp.int32)
    b16 = jnp.broadcast_to(b, (LANES,)).astype(jnp.float32)
    y = _sc_kernel(xt, ew, b16)
    return y.reshape(B, 1)
